# baseline (speedup 1.0000x reference)
import functools

import jax
import jax.numpy as jnp
from jax import lax
from jax.experimental import pallas as pl
from jax.experimental.pallas import tpu as pltpu
from jax.experimental.pallas import tpu_sc as plsc

N_USERS = 6000
D = 128
NC = 2
NS = 16
L = 16
NW = NC * NS
CHUNK = 64
U = 16
BN = 1024


def _mm2_body(x_ref, wl_ref, wr_ref, att_ref, xl_ref, xr_ref, sl_ref, sr_ref):
  x = x_ref[...]
  xl = jnp.dot(x, wl_ref[...], preferred_element_type=jnp.float32)
  xr = jnp.dot(x, wr_ref[...], preferred_element_type=jnp.float32)
  xl_ref[...] = xl
  xr_ref[...] = xr
  sl_ref[...] = 0.6 * jnp.dot(xl, att_ref[...], preferred_element_type=jnp.float32)
  sr_ref[...] = 0.6 * jnp.dot(xr, att_ref[...], preferred_element_type=jnp.float32)


def _mm2(x, wl, wr, att):
  n = x.shape[0]
  return pl.pallas_call(
      _mm2_body,
      grid=(n // BN,),
      in_specs=[
          pl.BlockSpec((BN, D), lambda i: (i, 0)),
          pl.BlockSpec((D, D), lambda i: (0, 0)),
          pl.BlockSpec((D, D), lambda i: (0, 0)),
          pl.BlockSpec((D, 1), lambda i: (0, 0)),
      ],
      out_specs=[
          pl.BlockSpec((BN, D), lambda i: (i, 0)),
          pl.BlockSpec((BN, D), lambda i: (i, 0)),
          pl.BlockSpec((BN, 1), lambda i: (i, 0)),
          pl.BlockSpec((BN, 1), lambda i: (i, 0)),
      ],
      out_shape=[jax.ShapeDtypeStruct((n, D), jnp.float32)] * 2
      + [jax.ShapeDtypeStruct((n, 1), jnp.float32)] * 2,
  )(x, wl, wr, att)


def _comb_mm2_body(acc_ref, s_ref, b_ref, res_ref, wl_ref, wr_ref, att_ref,
                   h_ref, xl_ref, xr_ref, sl_ref, sr_ref):
  s = jnp.sum(s_ref[...], axis=0) + 1e-16
  h = (acc_ref[0] + acc_ref[1]) / s[:, None] + b_ref[...] + res_ref[...]
  h = jnp.maximum(h, 0.0)
  h_ref[...] = h
  xl = jnp.dot(h, wl_ref[...], preferred_element_type=jnp.float32)
  xr = jnp.dot(h, wr_ref[...], preferred_element_type=jnp.float32)
  xl_ref[...] = xl
  xr_ref[...] = xr
  sl_ref[...] = 0.6 * jnp.dot(xl, att_ref[...], preferred_element_type=jnp.float32)
  sr_ref[...] = 0.6 * jnp.dot(xr, att_ref[...], preferred_element_type=jnp.float32)


def _comb_mm2(acc, s, b, res, wl, wr, att):
  n = res.shape[0]
  nw = s.shape[0]
  return pl.pallas_call(
      _comb_mm2_body,
      grid=(n // BN,),
      in_specs=[
          pl.BlockSpec((NC, BN, D), lambda i: (0, i, 0)),
          pl.BlockSpec((nw, BN), lambda i: (0, i)),
          pl.BlockSpec((1, D), lambda i: (0, 0)),
          pl.BlockSpec((BN, D), lambda i: (i, 0)),
          pl.BlockSpec((D, D), lambda i: (0, 0)),
          pl.BlockSpec((D, D), lambda i: (0, 0)),
          pl.BlockSpec((D, 1), lambda i: (0, 0)),
      ],
      out_specs=[
          pl.BlockSpec((BN, D), lambda i: (i, 0)),
          pl.BlockSpec((BN, D), lambda i: (i, 0)),
          pl.BlockSpec((BN, D), lambda i: (i, 0)),
          pl.BlockSpec((BN, 1), lambda i: (i, 0)),
          pl.BlockSpec((BN, 1), lambda i: (i, 0)),
      ],
      out_shape=[jax.ShapeDtypeStruct((n, D), jnp.float32)] * 3
      + [jax.ShapeDtypeStruct((n, 1), jnp.float32)] * 2,
  )(acc, s, b, res, wl, wr, att)


def _final_body(acc_ref, s_ref, b_ref, res_ref, y_ref):
  s = jnp.sum(s_ref[...], axis=0) + 1e-16
  y_ref[...] = (acc_ref[0] + acc_ref[1]) / s[:, None] + b_ref[...] + res_ref[...]


def _final(acc, s, b, res):
  n = res.shape[0]
  nw = s.shape[0]
  return pl.pallas_call(
      _final_body,
      grid=(n // BN,),
      in_specs=[
          pl.BlockSpec((NC, BN, D), lambda i: (0, i, 0)),
          pl.BlockSpec((nw, BN), lambda i: (0, i)),
          pl.BlockSpec((1, D), lambda i: (0, 0)),
          pl.BlockSpec((BN, D), lambda i: (i, 0)),
      ],
      out_specs=pl.BlockSpec((BN, D), lambda i: (i, 0)),
      out_shape=jax.ShapeDtypeStruct((n, D), jnp.float32),
  )(acc, s, b, res)


def _sc_mesh():
  return plsc.VectorSubcoreMesh(
      core_axis_name="c", subcore_axis_name="s", num_cores=NC, num_subcores=NS)


_SC_PARAMS = pltpu.CompilerParams(needs_layout_passes=False)


@functools.lru_cache(maxsize=None)
def _make_sc_edge(e_pad, n_pad, per_w):
  n_chunks = per_w // CHUNK
  n_pairs = n_chunks // 2
  rpt = n_pad // NS

  @functools.partial(
      pl.kernel,
      out_type=[
          jax.ShapeDtypeStruct((NW, n_pad), jnp.float32),
          jax.ShapeDtypeStruct((NC, n_pad, D), jnp.float32),
      ],
      mesh=_sc_mesh(),
      compiler_params=_SC_PARAMS,
      scratch_types=[
          pltpu.VMEM((CHUNK,), jnp.int32),
          pltpu.VMEM((CHUNK,), jnp.int32),
          pltpu.VMEM((CHUNK, D), jnp.float32),
          pltpu.VMEM((CHUNK, D), jnp.float32),
          pltpu.VMEM((CHUNK,), jnp.float32),
          pltpu.VMEM((CHUNK,), jnp.float32),
          pltpu.VMEM((CHUNK,), jnp.int32),
          pltpu.VMEM((CHUNK,), jnp.int32),
          pltpu.VMEM((CHUNK, D), jnp.float32),
          pltpu.VMEM((CHUNK, D), jnp.float32),
          pltpu.VMEM((CHUNK,), jnp.float32),
          pltpu.VMEM((CHUNK,), jnp.float32),
          pltpu.VMEM((D,), jnp.float32),
          pltpu.VMEM((n_pad,), jnp.float32),
          pltpu.VMEM_SHARED((n_pad, D), jnp.float32),
          pltpu.SemaphoreType.DMA,
          pltpu.SemaphoreType.DMA,
          pltpu.SemaphoreType.DMA,
          pltpu.SemaphoreType.DMA,
      ],
  )
  def sc_edge(xl_hbm, xr_hbm, sl_hbm, sr_hbm, src_hbm, dst_hbm, att_hbm,
              zero_nd_hbm,
              s_hbm, out_hbm,
              src_a, dst_a, xlr_a, xrr_a, slb_a, srb_a,
              src_b, dst_b, xlr_b, xrr_b, slb_b, srb_b,
              att_v, s_v, acc_sh, sem_a, sem_b, sem_sca, sem_scb):
    cid = lax.axis_index("c")
    sid = lax.axis_index("s")
    wid = cid * NS + sid
    pltpu.sync_copy(att_hbm, att_v)
    pltpu.sync_copy(zero_nd_hbm.at[pl.ds(sid * rpt, rpt)],
                    acc_sh.at[pl.ds(sid * rpt, rpt)])

    def zero_body(i, carry):
      s_v[pl.ds(i * L, L)] = jnp.zeros((L,), jnp.float32)
      return carry

    lax.fori_loop(0, n_pad // L, zero_body, 0)
    plsc.subcore_barrier()
    row16 = lax.iota(jnp.int32, L)
    att_q = [att_v[pl.ds(q * L, L)] for q in range(D // L)]

    A = (src_a, dst_a, xlr_a, xrr_a, slb_a, srb_a, sem_a, sem_sca)
    B = (src_b, dst_b, xlr_b, xrr_b, slb_b, srb_b, sem_b, sem_scb)

    def issue(ci, buf):
      srcb, dstb, xlb, xrb, slb, srb, sem, _ = buf
      base = wid * per_w + ci * CHUNK
      pltpu.sync_copy(src_hbm.at[pl.ds(base, CHUNK)], srcb)
      pltpu.sync_copy(dst_hbm.at[pl.ds(base, CHUNK)], dstb)
      pltpu.async_copy(xl_hbm.at[srcb], xlb, sem)
      pltpu.async_copy(xr_hbm.at[dstb], xrb, sem)
      pltpu.async_copy(sl_hbm.at[srcb], slb, sem)
      pltpu.async_copy(sr_hbm.at[dstb], srb, sem)

    def wait_gather(buf):
      srcb, dstb, xlb, xrb, slb, srb, sem, _ = buf
      pltpu.make_async_copy(xl_hbm.at[srcb], xlb, sem).wait()
      pltpu.make_async_copy(xr_hbm.at[dstb], xrb, sem).wait()
      pltpu.make_async_copy(sl_hbm.at[srcb], slb, sem).wait()
      pltpu.make_async_copy(sr_hbm.at[dstb], srb, sem).wait()

    def scatter(buf):
      _, dstb, xlb, _, _, _, _, scsem = buf
      pltpu.async_copy(xlb, acc_sh.at[dstb], scsem, add=True)

    def wait_scatter(buf):
      _, dstb, xlb, _, _, _, _, scsem = buf
      pltpu.make_async_copy(xlb, acc_sh.at[dstb], scsem).wait()

    def process(buf):
      srcb, dstb, xlb, xrb, slb, srb, _, _ = buf

      def group_body(g, carry2):
        lvec = slb[pl.ds(g * L, L)] + srb[pl.ds(g * L, L)]
        for u in range(L):
          j = g * L + u
          acc = None
          for q in range(D // L):
            t = xlb[j, pl.ds(q * L, L)] + xrb[j, pl.ds(q * L, L)]
            pr = att_q[q] * jnp.abs(t)
            acc = pr if acc is None else acc + pr
          lvec = lvec + jnp.where(row16 == u, jnp.sum(acc), 0.0)
        p16 = jnp.exp(lvec)
        plsc.addupdate_scatter(s_v, [dstb[pl.ds(g * L, L)]], p16)
        for u in range(L):
          j = g * L + u
          for q in range(D // L):
            sl = pl.ds(q * L, L)
            xlb[j, sl] = xlb[j, sl] * p16[u]
        return carry2

      lax.fori_loop(0, CHUNK // L, group_body, 0)

    issue(0, A)

    def pair_body(i, carry):
      @pl.when(i > 0)
      def _():
        wait_scatter(B)

      issue(2 * i + 1, B)
      wait_gather(A)
      process(A)
      scatter(A)
      wait_gather(B)
      process(B)

      @pl.when(i < n_pairs - 1)
      def _():
        wait_scatter(A)
        issue(2 * i + 2, A)

      scatter(B)
      return carry

    lax.fori_loop(0, n_pairs, pair_body, 0)
    wait_scatter(A)
    wait_scatter(B)
    pltpu.sync_copy(s_v, s_hbm.at[wid])
    plsc.subcore_barrier()
    pltpu.sync_copy(acc_sh.at[pl.ds(sid * rpt, rpt)],
                    out_hbm.at[cid, pl.ds(sid * rpt, rpt)])

  return sc_edge


def kernel(edge_index, emb, Wl1, Wr1, att1, b1, Wl2, Wr2, att2, b2):
  n = emb.shape[0]
  e2 = edge_index.shape[1] + n
  n_pad = ((n + BN) // BN) * BN
  per_w = -(-e2 // (NW * 2 * CHUNK)) * 2 * CHUNK
  e_pad = per_w * NW

  loop = jnp.arange(n, dtype=jnp.int32)
  pad_e = e_pad - e2
  src = jnp.concatenate(
      [edge_index[0], loop, jnp.zeros((pad_e,), jnp.int32)])
  dst = jnp.concatenate(
      [edge_index[1], loop, jnp.full((pad_e,), n, jnp.int32)])
  emb_p = jnp.pad(emb, ((0, n_pad - n), (0, 0)))
  zero_nd = jnp.zeros((n_pad, D), jnp.float32)
  b1r = b1.reshape(1, D)
  b2r = b2.reshape(1, D)
  att1c = att1.reshape(D, 1)
  att2c = att2.reshape(D, 1)
  att1s = 0.4 * att1
  att2s = 0.4 * att2

  sc_edge = _make_sc_edge(e_pad, n_pad, per_w)

  xl1, xr1, sl1, sr1 = _mm2(emb_p, Wl1, Wr1, att1c)
  s1, acc1 = sc_edge(xl1, xr1, sl1.reshape(n_pad), sr1.reshape(n_pad),
                     src, dst, att1s, zero_nd)
  h, xl2, xr2, sl2, sr2 = _comb_mm2(acc1, s1, b1r, emb_p, Wl2, Wr2, att2c)
  s2, acc2 = sc_edge(xl2, xr2, sl2.reshape(n_pad), sr2.reshape(n_pad),
                     src, dst, att2s, zero_nd)
  y = _final(acc2, s2, b2r, h)

  y = y[:n]
  return (y[:N_USERS], y[N_USERS:])


# speedup vs baseline: 1.0793x; 1.0793x over previous
import functools

import jax
import jax.numpy as jnp
from jax import lax
from jax.experimental import pallas as pl
from jax.experimental.pallas import tpu as pltpu
from jax.experimental.pallas import tpu_sc as plsc

N_USERS = 6000
D = 128
NC = 2
NS = 16
L = 16
NW = NC * NS
CHUNK = 64
U = 16
BN = 1024


def _mm2_body(x_ref, wl_ref, wr_ref, att_ref, xl_ref, xr_ref, sl_ref, sr_ref):
  x = x_ref[...]
  xl = jnp.dot(x, wl_ref[...], preferred_element_type=jnp.float32)
  xr = jnp.dot(x, wr_ref[...], preferred_element_type=jnp.float32)
  xl_ref[...] = xl
  xr_ref[...] = xr
  sl_ref[...] = 0.6 * jnp.dot(xl, att_ref[...], preferred_element_type=jnp.float32)
  sr_ref[...] = 0.6 * jnp.dot(xr, att_ref[...], preferred_element_type=jnp.float32)


def _mm2(x, wl, wr, att):
  n = x.shape[0]
  return pl.pallas_call(
      _mm2_body,
      grid=(n // BN,),
      in_specs=[
          pl.BlockSpec((BN, D), lambda i: (i, 0)),
          pl.BlockSpec((D, D), lambda i: (0, 0)),
          pl.BlockSpec((D, D), lambda i: (0, 0)),
          pl.BlockSpec((D, 1), lambda i: (0, 0)),
      ],
      out_specs=[
          pl.BlockSpec((BN, D), lambda i: (i, 0)),
          pl.BlockSpec((BN, D), lambda i: (i, 0)),
          pl.BlockSpec((BN, 1), lambda i: (i, 0)),
          pl.BlockSpec((BN, 1), lambda i: (i, 0)),
      ],
      out_shape=[jax.ShapeDtypeStruct((n, D), jnp.float32)] * 2
      + [jax.ShapeDtypeStruct((n, 1), jnp.float32)] * 2,
  )(x, wl, wr, att)


def _comb_mm2_body(acc_ref, s_ref, b_ref, res_ref, wl_ref, wr_ref, att_ref,
                   h_ref, xl_ref, xr_ref, sl_ref, sr_ref):
  s = jnp.sum(s_ref[...], axis=0) + 1e-16
  h = (acc_ref[0] + acc_ref[1]) / s[:, None] + b_ref[...] + res_ref[...]
  h = jnp.maximum(h, 0.0)
  h_ref[...] = h
  xl = jnp.dot(h, wl_ref[...], preferred_element_type=jnp.float32)
  xr = jnp.dot(h, wr_ref[...], preferred_element_type=jnp.float32)
  xl_ref[...] = xl
  xr_ref[...] = xr
  sl_ref[...] = 0.6 * jnp.dot(xl, att_ref[...], preferred_element_type=jnp.float32)
  sr_ref[...] = 0.6 * jnp.dot(xr, att_ref[...], preferred_element_type=jnp.float32)


def _comb_mm2(acc, s, b, res, wl, wr, att):
  n = res.shape[0]
  nw = s.shape[0]
  return pl.pallas_call(
      _comb_mm2_body,
      grid=(n // BN,),
      in_specs=[
          pl.BlockSpec((NC, BN, D), lambda i: (0, i, 0)),
          pl.BlockSpec((nw, BN), lambda i: (0, i)),
          pl.BlockSpec((1, D), lambda i: (0, 0)),
          pl.BlockSpec((BN, D), lambda i: (i, 0)),
          pl.BlockSpec((D, D), lambda i: (0, 0)),
          pl.BlockSpec((D, D), lambda i: (0, 0)),
          pl.BlockSpec((D, 1), lambda i: (0, 0)),
      ],
      out_specs=[
          pl.BlockSpec((BN, D), lambda i: (i, 0)),
          pl.BlockSpec((BN, D), lambda i: (i, 0)),
          pl.BlockSpec((BN, D), lambda i: (i, 0)),
          pl.BlockSpec((BN, 1), lambda i: (i, 0)),
          pl.BlockSpec((BN, 1), lambda i: (i, 0)),
      ],
      out_shape=[jax.ShapeDtypeStruct((n, D), jnp.float32)] * 3
      + [jax.ShapeDtypeStruct((n, 1), jnp.float32)] * 2,
  )(acc, s, b, res, wl, wr, att)


def _final_body(acc_ref, s_ref, b_ref, res_ref, y_ref):
  s = jnp.sum(s_ref[...], axis=0) + 1e-16
  y_ref[...] = (acc_ref[0] + acc_ref[1]) / s[:, None] + b_ref[...] + res_ref[...]


def _final(acc, s, b, res):
  n = res.shape[0]
  nw = s.shape[0]
  return pl.pallas_call(
      _final_body,
      grid=(n // BN,),
      in_specs=[
          pl.BlockSpec((NC, BN, D), lambda i: (0, i, 0)),
          pl.BlockSpec((nw, BN), lambda i: (0, i)),
          pl.BlockSpec((1, D), lambda i: (0, 0)),
          pl.BlockSpec((BN, D), lambda i: (i, 0)),
      ],
      out_specs=pl.BlockSpec((BN, D), lambda i: (i, 0)),
      out_shape=jax.ShapeDtypeStruct((n, D), jnp.float32),
  )(acc, s, b, res)


def _sc_mesh():
  return plsc.VectorSubcoreMesh(
      core_axis_name="c", subcore_axis_name="s", num_cores=NC, num_subcores=NS)


_SC_PARAMS = pltpu.CompilerParams(needs_layout_passes=False)


@functools.lru_cache(maxsize=None)
def _make_sc_edge(e_pad, n_pad, per_w):
  n_chunks = per_w // CHUNK
  n_pairs = n_chunks // 2
  rpt = n_pad // NS

  @functools.partial(
      pl.kernel,
      out_type=[
          jax.ShapeDtypeStruct((NW, n_pad), jnp.float32),
          jax.ShapeDtypeStruct((NC, n_pad, D), jnp.float32),
      ],
      mesh=_sc_mesh(),
      compiler_params=_SC_PARAMS,
      scratch_types=[
          pltpu.VMEM((CHUNK,), jnp.int32),
          pltpu.VMEM((CHUNK,), jnp.int32),
          pltpu.VMEM((CHUNK, D), jnp.float32),
          pltpu.VMEM((CHUNK, D), jnp.float32),
          pltpu.VMEM((CHUNK,), jnp.int32),
          pltpu.VMEM((CHUNK,), jnp.int32),
          pltpu.VMEM((CHUNK, D), jnp.float32),
          pltpu.VMEM((CHUNK, D), jnp.float32),
          pltpu.VMEM((D,), jnp.float32),
          pltpu.VMEM((n_pad,), jnp.float32),
          pltpu.VMEM_SHARED((n_pad, D), jnp.float32),
          pltpu.SemaphoreType.DMA,
          pltpu.SemaphoreType.DMA,
          pltpu.SemaphoreType.DMA,
          pltpu.SemaphoreType.DMA,
      ],
  )
  def sc_edge(xl_hbm, xr_hbm, src_hbm, dst_hbm, att_hbm,
              zero_nd_hbm,
              s_hbm, out_hbm,
              src_a, dst_a, xlr_a, xrr_a,
              src_b, dst_b, xlr_b, xrr_b,
              att_v, s_v, acc_sh, sem_a, sem_b, sem_sca, sem_scb):
    cid = lax.axis_index("c")
    sid = lax.axis_index("s")
    wid = cid * NS + sid
    pltpu.sync_copy(att_hbm, att_v)
    pltpu.sync_copy(zero_nd_hbm.at[pl.ds(sid * rpt, rpt)],
                    acc_sh.at[pl.ds(sid * rpt, rpt)])

    def zero_body(i, carry):
      s_v[pl.ds(i * L, L)] = jnp.zeros((L,), jnp.float32)
      return carry

    lax.fori_loop(0, n_pad // L, zero_body, 0)
    plsc.subcore_barrier()
    row16 = lax.iota(jnp.int32, L)
    att_q = [att_v[pl.ds(q * L, L)] for q in range(D // L)]

    A = (src_a, dst_a, xlr_a, xrr_a, sem_a, sem_sca)
    B = (src_b, dst_b, xlr_b, xrr_b, sem_b, sem_scb)

    def issue(ci, buf):
      srcb, dstb, xlb, xrb, sem, _ = buf
      base = wid * per_w + ci * CHUNK
      pltpu.sync_copy(src_hbm.at[pl.ds(base, CHUNK)], srcb)
      pltpu.sync_copy(dst_hbm.at[pl.ds(base, CHUNK)], dstb)
      pltpu.async_copy(xl_hbm.at[srcb], xlb, sem)
      pltpu.async_copy(xr_hbm.at[dstb], xrb, sem)

    def wait_gather(buf):
      srcb, dstb, xlb, xrb, sem, _ = buf
      pltpu.make_async_copy(xl_hbm.at[srcb], xlb, sem).wait()
      pltpu.make_async_copy(xr_hbm.at[dstb], xrb, sem).wait()

    def scatter(buf):
      _, dstb, xlb, _, _, scsem = buf
      pltpu.async_copy(xlb, acc_sh.at[dstb], scsem, add=True)

    def wait_scatter(buf):
      _, dstb, xlb, _, _, scsem = buf
      pltpu.make_async_copy(xlb, acc_sh.at[dstb], scsem).wait()

    def process(buf):
      srcb, dstb, xlb, xrb, _, _ = buf

      def group_body(g, carry2):
        lvec = jnp.zeros((L,), jnp.float32)
        for u in range(L):
          j = g * L + u
          acc = None
          for q in range(D // L):
            t = xlb[j, pl.ds(q * L, L)] + xrb[j, pl.ds(q * L, L)]
            t = jnp.maximum(t, 0.2 * t)
            pr = att_q[q] * t
            acc = pr if acc is None else acc + pr
          lvec = lvec + jnp.where(row16 == u, jnp.sum(acc), 0.0)
        p16 = jnp.exp(lvec)
        plsc.addupdate_scatter(s_v, [dstb[pl.ds(g * L, L)]], p16)
        for u in range(L):
          j = g * L + u
          for q in range(D // L):
            sl = pl.ds(q * L, L)
            xlb[j, sl] = xlb[j, sl] * p16[u]
        return carry2

      lax.fori_loop(0, CHUNK // L, group_body, 0)

    issue(0, A)

    def pair_body(i, carry):
      @pl.when(i > 0)
      def _():
        wait_scatter(B)

      issue(2 * i + 1, B)
      wait_gather(A)
      process(A)
      scatter(A)
      wait_gather(B)
      process(B)

      @pl.when(i < n_pairs - 1)
      def _():
        wait_scatter(A)
        issue(2 * i + 2, A)

      scatter(B)
      return carry

    lax.fori_loop(0, n_pairs, pair_body, 0)
    wait_scatter(A)
    wait_scatter(B)
    pltpu.sync_copy(s_v, s_hbm.at[wid])
    plsc.subcore_barrier()
    pltpu.sync_copy(acc_sh.at[pl.ds(sid * rpt, rpt)],
                    out_hbm.at[cid, pl.ds(sid * rpt, rpt)])

  return sc_edge


def kernel(edge_index, emb, Wl1, Wr1, att1, b1, Wl2, Wr2, att2, b2):
  n = emb.shape[0]
  e2 = edge_index.shape[1] + n
  n_pad = ((n + BN) // BN) * BN
  per_w = -(-e2 // (NW * 2 * CHUNK)) * 2 * CHUNK
  e_pad = per_w * NW

  loop = jnp.arange(n, dtype=jnp.int32)
  pad_e = e_pad - e2
  src = jnp.concatenate(
      [edge_index[0], loop, jnp.zeros((pad_e,), jnp.int32)])
  dst = jnp.concatenate(
      [edge_index[1], loop, jnp.full((pad_e,), n, jnp.int32)])
  emb_p = jnp.pad(emb, ((0, n_pad - n), (0, 0)))
  zero_nd = jnp.zeros((n_pad, D), jnp.float32)
  b1r = b1.reshape(1, D)
  b2r = b2.reshape(1, D)
  att1c = att1.reshape(D, 1)
  att2c = att2.reshape(D, 1)
  att1s = att1
  att2s = att2

  sc_edge = _make_sc_edge(e_pad, n_pad, per_w)

  xl1, xr1, sl1, sr1 = _mm2(emb_p, Wl1, Wr1, att1c)
  s1, acc1 = sc_edge(xl1, xr1, src, dst, att1s, zero_nd)
  h, xl2, xr2, sl2, sr2 = _comb_mm2(acc1, s1, b1r, emb_p, Wl2, Wr2, att2c)
  s2, acc2 = sc_edge(xl2, xr2, src, dst, att2s, zero_nd)
  y = _final(acc2, s2, b2r, h)

  y = y[:n]
  return (y[:N_USERS], y[N_USERS:])


# speedup vs baseline: 1.1315x; 1.0484x over previous
import functools

import jax
import jax.numpy as jnp
from jax import lax
from jax.experimental import pallas as pl
from jax.experimental.pallas import tpu as pltpu
from jax.experimental.pallas import tpu_sc as plsc

N_USERS = 6000
D = 128
NC = 2
NS = 16
L = 16
NW = NC * NS
CHUNK = 64
U = 16
BN = 1024


def _mm2_body(x_ref, wl_ref, wr_ref, att_ref, xl_ref, xr_ref, sl_ref, sr_ref):
  x = x_ref[...]
  xl = jnp.dot(x, wl_ref[...], preferred_element_type=jnp.float32)
  xr = jnp.dot(x, wr_ref[...], preferred_element_type=jnp.float32)
  xl_ref[...] = xl
  xr_ref[...] = xr
  sl_ref[...] = 0.6 * jnp.dot(xl, att_ref[...], preferred_element_type=jnp.float32)
  sr_ref[...] = 0.6 * jnp.dot(xr, att_ref[...], preferred_element_type=jnp.float32)


def _mm2(x, wl, wr, att):
  n = x.shape[0]
  return pl.pallas_call(
      _mm2_body,
      grid=(n // BN,),
      in_specs=[
          pl.BlockSpec((BN, D), lambda i: (i, 0)),
          pl.BlockSpec((D, D), lambda i: (0, 0)),
          pl.BlockSpec((D, D), lambda i: (0, 0)),
          pl.BlockSpec((D, 1), lambda i: (0, 0)),
      ],
      out_specs=[
          pl.BlockSpec((BN, D), lambda i: (i, 0)),
          pl.BlockSpec((BN, D), lambda i: (i, 0)),
          pl.BlockSpec((BN, 1), lambda i: (i, 0)),
          pl.BlockSpec((BN, 1), lambda i: (i, 0)),
      ],
      out_shape=[jax.ShapeDtypeStruct((n, D), jnp.float32)] * 2
      + [jax.ShapeDtypeStruct((n, 1), jnp.float32)] * 2,
  )(x, wl, wr, att)


def _comb_mm2_body(acc_ref, s_ref, b_ref, res_ref, wl_ref, wr_ref, att_ref,
                   h_ref, xl_ref, xr_ref, sl_ref, sr_ref):
  s = jnp.sum(s_ref[...], axis=0) + 1e-16
  h = (acc_ref[0] + acc_ref[1]) / s[:, None] + b_ref[...] + res_ref[...]
  h = jnp.maximum(h, 0.0)
  h_ref[...] = h
  xl = jnp.dot(h, wl_ref[...], preferred_element_type=jnp.float32)
  xr = jnp.dot(h, wr_ref[...], preferred_element_type=jnp.float32)
  xl_ref[...] = xl
  xr_ref[...] = xr
  sl_ref[...] = 0.6 * jnp.dot(xl, att_ref[...], preferred_element_type=jnp.float32)
  sr_ref[...] = 0.6 * jnp.dot(xr, att_ref[...], preferred_element_type=jnp.float32)


def _comb_mm2(acc, s, b, res, wl, wr, att):
  n = res.shape[0]
  nw = s.shape[0]
  return pl.pallas_call(
      _comb_mm2_body,
      grid=(n // BN,),
      in_specs=[
          pl.BlockSpec((NC, BN, D), lambda i: (0, i, 0)),
          pl.BlockSpec((nw, BN), lambda i: (0, i)),
          pl.BlockSpec((1, D), lambda i: (0, 0)),
          pl.BlockSpec((BN, D), lambda i: (i, 0)),
          pl.BlockSpec((D, D), lambda i: (0, 0)),
          pl.BlockSpec((D, D), lambda i: (0, 0)),
          pl.BlockSpec((D, 1), lambda i: (0, 0)),
      ],
      out_specs=[
          pl.BlockSpec((BN, D), lambda i: (i, 0)),
          pl.BlockSpec((BN, D), lambda i: (i, 0)),
          pl.BlockSpec((BN, D), lambda i: (i, 0)),
          pl.BlockSpec((BN, 1), lambda i: (i, 0)),
          pl.BlockSpec((BN, 1), lambda i: (i, 0)),
      ],
      out_shape=[jax.ShapeDtypeStruct((n, D), jnp.float32)] * 3
      + [jax.ShapeDtypeStruct((n, 1), jnp.float32)] * 2,
  )(acc, s, b, res, wl, wr, att)


def _final_body(acc_ref, s_ref, b_ref, res_ref, y_ref):
  s = jnp.sum(s_ref[...], axis=0) + 1e-16
  y_ref[...] = (acc_ref[0] + acc_ref[1]) / s[:, None] + b_ref[...] + res_ref[...]


def _final(acc, s, b, res):
  n = res.shape[0]
  nw = s.shape[0]
  return pl.pallas_call(
      _final_body,
      grid=(n // BN,),
      in_specs=[
          pl.BlockSpec((NC, BN, D), lambda i: (0, i, 0)),
          pl.BlockSpec((nw, BN), lambda i: (0, i)),
          pl.BlockSpec((1, D), lambda i: (0, 0)),
          pl.BlockSpec((BN, D), lambda i: (i, 0)),
      ],
      out_specs=pl.BlockSpec((BN, D), lambda i: (i, 0)),
      out_shape=jax.ShapeDtypeStruct((n, D), jnp.float32),
  )(acc, s, b, res)


def _sc_mesh():
  return plsc.VectorSubcoreMesh(
      core_axis_name="c", subcore_axis_name="s", num_cores=NC, num_subcores=NS)


_SC_PARAMS = pltpu.CompilerParams(needs_layout_passes=False)


@functools.lru_cache(maxsize=None)
def _make_sc_edge(e_pad, n_pad, per_w):
  n_chunks = per_w // CHUNK
  n_pairs = n_chunks // 2
  rpt = n_pad // NS

  @functools.partial(
      pl.kernel,
      out_type=[
          jax.ShapeDtypeStruct((NW, n_pad), jnp.float32),
          jax.ShapeDtypeStruct((NC, n_pad, D), jnp.float32),
      ],
      mesh=_sc_mesh(),
      compiler_params=_SC_PARAMS,
      scratch_types=[
          pltpu.VMEM((CHUNK,), jnp.int32),
          pltpu.VMEM((CHUNK,), jnp.int32),
          pltpu.VMEM((CHUNK, D), jnp.float32),
          pltpu.VMEM((CHUNK, D), jnp.float32),
          pltpu.VMEM((CHUNK,), jnp.int32),
          pltpu.VMEM((CHUNK,), jnp.int32),
          pltpu.VMEM((CHUNK, D), jnp.float32),
          pltpu.VMEM((CHUNK, D), jnp.float32),
          pltpu.VMEM((D,), jnp.float32),
          pltpu.VMEM((n_pad,), jnp.float32),
          pltpu.VMEM_SHARED((n_pad, D), jnp.float32),
          pltpu.SemaphoreType.DMA,
          pltpu.SemaphoreType.DMA,
      ],
  )
  def sc_edge(xl_hbm, xr_hbm, src_hbm, dst_hbm, att_hbm,
              zero_nd_hbm,
              s_hbm, out_hbm,
              src_a, dst_a, xlr_a, xrr_a,
              src_b, dst_b, xlr_b, xrr_b,
              att_v, s_v, acc_sh, sem_a, sem_b):
    cid = lax.axis_index("c")
    sid = lax.axis_index("s")
    wid = cid * NS + sid
    pltpu.sync_copy(att_hbm, att_v)
    pltpu.sync_copy(zero_nd_hbm.at[pl.ds(sid * rpt, rpt)],
                    acc_sh.at[pl.ds(sid * rpt, rpt)])

    def zero_body(i, carry):
      s_v[pl.ds(i * L, L)] = jnp.zeros((L,), jnp.float32)
      return carry

    lax.fori_loop(0, n_pad // L, zero_body, 0)
    plsc.subcore_barrier()
    row16 = lax.iota(jnp.int32, L)
    att_q = [att_v[pl.ds(q * L, L)] for q in range(D // L)]

    A = (src_a, dst_a, xlr_a, xrr_a, sem_a)
    B = (src_b, dst_b, xlr_b, xrr_b, sem_b)

    def issue(ci, buf):
      srcb, dstb, xlb, xrb, sem = buf
      base = wid * per_w + ci * CHUNK
      pltpu.sync_copy(src_hbm.at[pl.ds(base, CHUNK)], srcb)
      pltpu.sync_copy(dst_hbm.at[pl.ds(base, CHUNK)], dstb)
      pltpu.async_copy(xl_hbm.at[srcb], xlb, sem)
      pltpu.async_copy(xr_hbm.at[dstb], xrb, sem)

    def wait_gather(buf):
      srcb, dstb, xlb, xrb, sem = buf
      pltpu.make_async_copy(xl_hbm.at[srcb], xlb, sem).wait()
      pltpu.make_async_copy(xr_hbm.at[dstb], xrb, sem).wait()

    def process(buf):
      srcb, dstb, xlb, xrb, _ = buf

      def group_body(g, carry2):
        lvec = jnp.zeros((L,), jnp.float32)
        for u in range(L):
          j = g * L + u
          acc = None
          for q in range(D // L):
            t = xlb[j, pl.ds(q * L, L)] + xrb[j, pl.ds(q * L, L)]
            t = jnp.maximum(t, 0.2 * t)
            pr = att_q[q] * t
            acc = pr if acc is None else acc + pr
          lvec = lvec + jnp.where(row16 == u, jnp.sum(acc), 0.0)
        p16 = jnp.exp(lvec)
        plsc.addupdate_scatter(s_v, [dstb[pl.ds(g * L, L)]], p16)
        for u in range(L):
          j = g * L + u
          for q in range(D // L):
            sl = pl.ds(q * L, L)
            xlb[j, sl] = xlb[j, sl] * p16[u]
        return carry2

      lax.fori_loop(0, CHUNK // L, group_body, 0)
      dstb2 = buf[1]
      pltpu.sync_copy(buf[2], acc_sh.at[dstb2], add=True)

    issue(0, A)

    def pair_body(i, carry):
      issue(2 * i + 1, B)
      wait_gather(A)
      process(A)

      @pl.when(i < n_pairs - 1)
      def _():
        issue(2 * i + 2, A)

      wait_gather(B)
      process(B)
      return carry

    lax.fori_loop(0, n_pairs, pair_body, 0)
    pltpu.sync_copy(s_v, s_hbm.at[wid])
    plsc.subcore_barrier()
    pltpu.sync_copy(acc_sh.at[pl.ds(sid * rpt, rpt)],
                    out_hbm.at[cid, pl.ds(sid * rpt, rpt)])

  return sc_edge


def kernel(edge_index, emb, Wl1, Wr1, att1, b1, Wl2, Wr2, att2, b2):
  n = emb.shape[0]
  e2 = edge_index.shape[1] + n
  n_pad = ((n + BN) // BN) * BN
  per_w = -(-e2 // (NW * 2 * CHUNK)) * 2 * CHUNK
  e_pad = per_w * NW

  loop = jnp.arange(n, dtype=jnp.int32)
  pad_e = e_pad - e2
  src = jnp.concatenate(
      [edge_index[0], loop, jnp.zeros((pad_e,), jnp.int32)])
  dst = jnp.concatenate(
      [edge_index[1], loop, jnp.full((pad_e,), n, jnp.int32)])
  emb_p = jnp.pad(emb, ((0, n_pad - n), (0, 0)))
  zero_nd = jnp.zeros((n_pad, D), jnp.float32)
  b1r = b1.reshape(1, D)
  b2r = b2.reshape(1, D)
  att1c = att1.reshape(D, 1)
  att2c = att2.reshape(D, 1)
  att1s = att1
  att2s = att2

  sc_edge = _make_sc_edge(e_pad, n_pad, per_w)

  xl1, xr1, sl1, sr1 = _mm2(emb_p, Wl1, Wr1, att1c)
  s1, acc1 = sc_edge(xl1, xr1, src, dst, att1s, zero_nd)
  h, xl2, xr2, sl2, sr2 = _comb_mm2(acc1, s1, b1r, emb_p, Wl2, Wr2, att2c)
  s2, acc2 = sc_edge(xl2, xr2, src, dst, att2s, zero_nd)
  y = _final(acc2, s2, b2r, h)

  y = y[:n]
  return (y[:N_USERS], y[N_USERS:])


# speedup vs baseline: 1.1580x; 1.0234x over previous
import functools

import jax
import jax.numpy as jnp
from jax import lax
from jax.experimental import pallas as pl
from jax.experimental.pallas import tpu as pltpu
from jax.experimental.pallas import tpu_sc as plsc

N_USERS = 6000
D = 128
NC = 2
NS = 16
L = 16
NW = NC * NS
CHUNK = 64
U = 16
BN = 1024


def _mm2_body(x_ref, wl_ref, wr_ref, xl_ref, xr_ref):
  x = x_ref[...]
  xl_ref[...] = jnp.dot(x, wl_ref[...], preferred_element_type=jnp.float32)
  xr_ref[...] = jnp.dot(x, wr_ref[...], preferred_element_type=jnp.float32)


def _mm2(x, wl, wr):
  n = x.shape[0]
  return pl.pallas_call(
      _mm2_body,
      grid=(n // BN,),
      in_specs=[
          pl.BlockSpec((BN, D), lambda i: (i, 0)),
          pl.BlockSpec((D, D), lambda i: (0, 0)),
          pl.BlockSpec((D, D), lambda i: (0, 0)),
      ],
      out_specs=[
          pl.BlockSpec((BN, D), lambda i: (i, 0)),
          pl.BlockSpec((BN, D), lambda i: (i, 0)),
      ],
      out_shape=[jax.ShapeDtypeStruct((n, D), jnp.float32)] * 2,
  )(x, wl, wr)


def _comb_mm2_body(acc_ref, s_ref, b_ref, res_ref, wl_ref, wr_ref,
                   h_ref, xl_ref, xr_ref):
  s = jnp.sum(s_ref[...], axis=0) + 1e-16
  h = (acc_ref[0] + acc_ref[1]) / s[:, None] + b_ref[...] + res_ref[...]
  h = jnp.maximum(h, 0.0)
  h_ref[...] = h
  xl_ref[...] = jnp.dot(h, wl_ref[...], preferred_element_type=jnp.float32)
  xr_ref[...] = jnp.dot(h, wr_ref[...], preferred_element_type=jnp.float32)


def _comb_mm2(acc, s, b, res, wl, wr):
  n = res.shape[0]
  nw = s.shape[0]
  return pl.pallas_call(
      _comb_mm2_body,
      grid=(n // BN,),
      in_specs=[
          pl.BlockSpec((NC, BN, D), lambda i: (0, i, 0)),
          pl.BlockSpec((nw, BN), lambda i: (0, i)),
          pl.BlockSpec((1, D), lambda i: (0, 0)),
          pl.BlockSpec((BN, D), lambda i: (i, 0)),
          pl.BlockSpec((D, D), lambda i: (0, 0)),
          pl.BlockSpec((D, D), lambda i: (0, 0)),
      ],
      out_specs=[
          pl.BlockSpec((BN, D), lambda i: (i, 0)),
          pl.BlockSpec((BN, D), lambda i: (i, 0)),
          pl.BlockSpec((BN, D), lambda i: (i, 0)),
      ],
      out_shape=[jax.ShapeDtypeStruct((n, D), jnp.float32)] * 3,
  )(acc, s, b, res, wl, wr)


def _final_body(acc_ref, s_ref, b_ref, res_ref, y_ref):
  s = jnp.sum(s_ref[...], axis=0) + 1e-16
  y_ref[...] = (acc_ref[0] + acc_ref[1]) / s[:, None] + b_ref[...] + res_ref[...]


def _final(acc, s, b, res):
  n = res.shape[0]
  nw = s.shape[0]
  return pl.pallas_call(
      _final_body,
      grid=(n // BN,),
      in_specs=[
          pl.BlockSpec((NC, BN, D), lambda i: (0, i, 0)),
          pl.BlockSpec((nw, BN), lambda i: (0, i)),
          pl.BlockSpec((1, D), lambda i: (0, 0)),
          pl.BlockSpec((BN, D), lambda i: (i, 0)),
      ],
      out_specs=pl.BlockSpec((BN, D), lambda i: (i, 0)),
      out_shape=jax.ShapeDtypeStruct((n, D), jnp.float32),
  )(acc, s, b, res)


def _sc_mesh():
  return plsc.VectorSubcoreMesh(
      core_axis_name="c", subcore_axis_name="s", num_cores=NC, num_subcores=NS)


_SC_PARAMS = pltpu.CompilerParams(needs_layout_passes=False)


@functools.lru_cache(maxsize=None)
def _make_sc_edge(e_pad, n_pad, per_w):
  n_chunks = per_w // CHUNK
  n_pairs = n_chunks // 2
  rpt = n_pad // NS

  @functools.partial(
      pl.kernel,
      out_type=[
          jax.ShapeDtypeStruct((NW, n_pad), jnp.float32),
          jax.ShapeDtypeStruct((NC, n_pad, D), jnp.float32),
      ],
      mesh=_sc_mesh(),
      compiler_params=_SC_PARAMS,
      scratch_types=[
          pltpu.VMEM((CHUNK,), jnp.int32),
          pltpu.VMEM((CHUNK,), jnp.int32),
          pltpu.VMEM((CHUNK, D), jnp.float32),
          pltpu.VMEM((CHUNK, D), jnp.float32),
          pltpu.VMEM((CHUNK,), jnp.int32),
          pltpu.VMEM((CHUNK,), jnp.int32),
          pltpu.VMEM((CHUNK, D), jnp.float32),
          pltpu.VMEM((CHUNK, D), jnp.float32),
          pltpu.VMEM((D,), jnp.float32),
          pltpu.VMEM((n_pad,), jnp.float32),
          pltpu.VMEM_SHARED((n_pad, D), jnp.float32),
          pltpu.SemaphoreType.DMA,
          pltpu.SemaphoreType.DMA,
      ],
  )
  def sc_edge(xl_hbm, xr_hbm, src_hbm, dst_hbm, att_hbm,
              zero_nd_hbm,
              s_hbm, out_hbm,
              src_a, dst_a, xlr_a, xrr_a,
              src_b, dst_b, xlr_b, xrr_b,
              att_v, s_v, acc_sh, sem_a, sem_b):
    cid = lax.axis_index("c")
    sid = lax.axis_index("s")
    wid = cid * NS + sid
    pltpu.sync_copy(att_hbm, att_v)
    pltpu.sync_copy(zero_nd_hbm.at[pl.ds(sid * rpt, rpt)],
                    acc_sh.at[pl.ds(sid * rpt, rpt)])

    def zero_body(i, carry):
      s_v[pl.ds(i * L, L)] = jnp.zeros((L,), jnp.float32)
      return carry

    lax.fori_loop(0, n_pad // L, zero_body, 0)
    plsc.subcore_barrier()
    row16 = lax.iota(jnp.int32, L)
    att_q = [att_v[pl.ds(q * L, L)] for q in range(D // L)]

    A = (src_a, dst_a, xlr_a, xrr_a, sem_a)
    B = (src_b, dst_b, xlr_b, xrr_b, sem_b)

    def issue(ci, buf):
      srcb, dstb, xlb, xrb, sem = buf
      base = wid * per_w + ci * CHUNK
      pltpu.sync_copy(src_hbm.at[pl.ds(base, CHUNK)], srcb)
      pltpu.sync_copy(dst_hbm.at[pl.ds(base, CHUNK)], dstb)
      pltpu.async_copy(xl_hbm.at[srcb], xlb, sem)
      pltpu.async_copy(xr_hbm.at[dstb], xrb, sem)

    def wait_gather(buf):
      srcb, dstb, xlb, xrb, sem = buf
      pltpu.make_async_copy(xl_hbm.at[srcb], xlb, sem).wait()
      pltpu.make_async_copy(xr_hbm.at[dstb], xrb, sem).wait()

    def process(buf):
      srcb, dstb, xlb, xrb, _ = buf

      def group_body(g, carry2):
        p16 = jnp.zeros((L,), jnp.float32)
        for u in range(L):
          j = g * L + u
          acc = None
          xs = []
          for q in range(D // L):
            a = xlb[j, pl.ds(q * L, L)]
            xs.append(a)
            t = a + xrb[j, pl.ds(q * L, L)]
            t = jnp.maximum(t, 0.2 * t)
            pr = att_q[q] * t
            acc = pr if acc is None else acc + pr
          pb = jnp.exp(jnp.full((L,), jnp.sum(acc), jnp.float32))
          p16 = jnp.where(row16 == u, pb, p16)
          for q in range(D // L):
            xlb[j, pl.ds(q * L, L)] = xs[q] * pb
        plsc.addupdate_scatter(s_v, [dstb[pl.ds(g * L, L)]], p16)
        return carry2

      lax.fori_loop(0, CHUNK // L, group_body, 0)
      dstb2 = buf[1]
      pltpu.sync_copy(buf[2], acc_sh.at[dstb2], add=True)

    issue(0, A)

    def pair_body(i, carry):
      issue(2 * i + 1, B)
      wait_gather(A)
      process(A)

      @pl.when(i < n_pairs - 1)
      def _():
        issue(2 * i + 2, A)

      wait_gather(B)
      process(B)
      return carry

    lax.fori_loop(0, n_pairs, pair_body, 0)
    pltpu.sync_copy(s_v, s_hbm.at[wid])
    plsc.subcore_barrier()
    pltpu.sync_copy(acc_sh.at[pl.ds(sid * rpt, rpt)],
                    out_hbm.at[cid, pl.ds(sid * rpt, rpt)])

  return sc_edge


def kernel(edge_index, emb, Wl1, Wr1, att1, b1, Wl2, Wr2, att2, b2):
  n = emb.shape[0]
  e2 = edge_index.shape[1] + n
  n_pad = ((n + BN) // BN) * BN
  per_w = -(-e2 // (NW * 2 * CHUNK)) * 2 * CHUNK
  e_pad = per_w * NW

  loop = jnp.arange(n, dtype=jnp.int32)
  pad_e = e_pad - e2
  src = jnp.concatenate(
      [edge_index[0], loop, jnp.zeros((pad_e,), jnp.int32)])
  dst = jnp.concatenate(
      [edge_index[1], loop, jnp.full((pad_e,), n, jnp.int32)])
  emb_p = jnp.pad(emb, ((0, n_pad - n), (0, 0)))
  zero_nd = jnp.zeros((n_pad, D), jnp.float32)
  b1r = b1.reshape(1, D)
  b2r = b2.reshape(1, D)
  sc_edge = _make_sc_edge(e_pad, n_pad, per_w)

  xl1, xr1 = _mm2(emb_p, Wl1, Wr1)
  s1, acc1 = sc_edge(xl1, xr1, src, dst, att1, zero_nd)
  h, xl2, xr2 = _comb_mm2(acc1, s1, b1r, emb_p, Wl2, Wr2)
  s2, acc2 = sc_edge(xl2, xr2, src, dst, att2, zero_nd)
  y = _final(acc2, s2, b2r, h)

  y = y[:n]
  return (y[:N_USERS], y[N_USERS:])


# speedup vs baseline: 1.2658x; 1.0931x over previous
import functools

import jax
import jax.numpy as jnp
from jax import lax
from jax.experimental import pallas as pl
from jax.experimental.pallas import tpu as pltpu
from jax.experimental.pallas import tpu_sc as plsc

N_USERS = 6000
D = 128
NC = 2
NS = 16
L = 16
NW = NC * NS
CHUNK = 64
U = 16
BN = 1024


def _mm2_body(x_ref, wl_ref, wr_ref, xl_ref, xr_ref):
  x = x_ref[...]
  xl_ref[...] = jnp.dot(x, wl_ref[...], preferred_element_type=jnp.float32)
  xr_ref[...] = jnp.dot(x, wr_ref[...], preferred_element_type=jnp.float32)


def _mm2(x, wl, wr):
  n = x.shape[0]
  return pl.pallas_call(
      _mm2_body,
      grid=(n // BN,),
      in_specs=[
          pl.BlockSpec((BN, D), lambda i: (i, 0)),
          pl.BlockSpec((D, D), lambda i: (0, 0)),
          pl.BlockSpec((D, D), lambda i: (0, 0)),
      ],
      out_specs=[
          pl.BlockSpec((BN, D), lambda i: (i, 0)),
          pl.BlockSpec((BN, D), lambda i: (i, 0)),
      ],
      out_shape=[jax.ShapeDtypeStruct((n, D), jnp.float32)] * 2,
  )(x, wl, wr)


def _comb_mm2_body(acc_ref, s_ref, b_ref, res_ref, wl_ref, wr_ref,
                   h_ref, xl_ref, xr_ref):
  s = jnp.sum(s_ref[...], axis=0) + 1e-16
  h = (acc_ref[0] + acc_ref[1]) / s[:, None] + b_ref[...] + res_ref[...]
  h = jnp.maximum(h, 0.0)
  h_ref[...] = h
  xl_ref[...] = jnp.dot(h, wl_ref[...], preferred_element_type=jnp.float32)
  xr_ref[...] = jnp.dot(h, wr_ref[...], preferred_element_type=jnp.float32)


def _comb_mm2(acc, s, b, res, wl, wr):
  n = res.shape[0]
  nw = s.shape[0]
  return pl.pallas_call(
      _comb_mm2_body,
      grid=(n // BN,),
      in_specs=[
          pl.BlockSpec((NC, BN, D), lambda i: (0, i, 0)),
          pl.BlockSpec((nw, BN), lambda i: (0, i)),
          pl.BlockSpec((1, D), lambda i: (0, 0)),
          pl.BlockSpec((BN, D), lambda i: (i, 0)),
          pl.BlockSpec((D, D), lambda i: (0, 0)),
          pl.BlockSpec((D, D), lambda i: (0, 0)),
      ],
      out_specs=[
          pl.BlockSpec((BN, D), lambda i: (i, 0)),
          pl.BlockSpec((BN, D), lambda i: (i, 0)),
          pl.BlockSpec((BN, D), lambda i: (i, 0)),
      ],
      out_shape=[jax.ShapeDtypeStruct((n, D), jnp.float32)] * 3,
  )(acc, s, b, res, wl, wr)


def _final_body(acc_ref, s_ref, b_ref, res_ref, y_ref):
  s = jnp.sum(s_ref[...], axis=0) + 1e-16
  y_ref[...] = (acc_ref[0] + acc_ref[1]) / s[:, None] + b_ref[...] + res_ref[...]


def _final(acc, s, b, res):
  n = res.shape[0]
  nw = s.shape[0]
  return pl.pallas_call(
      _final_body,
      grid=(n // BN,),
      in_specs=[
          pl.BlockSpec((NC, BN, D), lambda i: (0, i, 0)),
          pl.BlockSpec((nw, BN), lambda i: (0, i)),
          pl.BlockSpec((1, D), lambda i: (0, 0)),
          pl.BlockSpec((BN, D), lambda i: (i, 0)),
      ],
      out_specs=pl.BlockSpec((BN, D), lambda i: (i, 0)),
      out_shape=jax.ShapeDtypeStruct((n, D), jnp.float32),
  )(acc, s, b, res)


def _sc_mesh():
  return plsc.VectorSubcoreMesh(
      core_axis_name="c", subcore_axis_name="s", num_cores=NC, num_subcores=NS)


_SC_PARAMS = pltpu.CompilerParams(needs_layout_passes=False)


@functools.lru_cache(maxsize=None)
def _make_sc_edge(e_pad, n_pad, per_w):
  n_chunks = per_w // CHUNK
  n_pairs = n_chunks // 2
  rpt = n_pad // NS

  @functools.partial(
      pl.kernel,
      out_type=[
          jax.ShapeDtypeStruct((NW, n_pad), jnp.float32),
          jax.ShapeDtypeStruct((NC, n_pad, D), jnp.float32),
      ],
      mesh=_sc_mesh(),
      compiler_params=_SC_PARAMS,
      scratch_types=[
          pltpu.VMEM((2, CHUNK), jnp.int32),
          pltpu.VMEM((CHUNK, D), jnp.float32),
          pltpu.VMEM((CHUNK, D), jnp.float32),
          pltpu.VMEM((2, CHUNK), jnp.int32),
          pltpu.VMEM((CHUNK, D), jnp.float32),
          pltpu.VMEM((CHUNK, D), jnp.float32),
          pltpu.VMEM((D,), jnp.float32),
          pltpu.VMEM((n_pad,), jnp.float32),
          pltpu.VMEM_SHARED((n_pad, D), jnp.float32),
          pltpu.SemaphoreType.DMA,
          pltpu.SemaphoreType.DMA,
      ],
  )
  def sc_edge(xl_hbm, xr_hbm, idx_hbm, att_hbm,
              zero_nd_hbm,
              s_hbm, out_hbm,
              idx_a, xlr_a, xrr_a,
              idx_b, xlr_b, xrr_b,
              att_v, s_v, acc_sh, sem_a, sem_b):
    cid = lax.axis_index("c")
    sid = lax.axis_index("s")
    wid = cid * NS + sid
    pltpu.sync_copy(att_hbm, att_v)
    pltpu.sync_copy(zero_nd_hbm.at[pl.ds(sid * rpt, rpt)],
                    acc_sh.at[pl.ds(sid * rpt, rpt)])

    def zero_body(i, carry):
      s_v[pl.ds(i * L, L)] = jnp.zeros((L,), jnp.float32)
      return carry

    lax.fori_loop(0, n_pad // L, zero_body, 0)
    plsc.subcore_barrier()
    row16 = lax.iota(jnp.int32, L)
    att_q = [att_v[pl.ds(q * L, L)] for q in range(D // L)]

    A = (idx_a, xlr_a, xrr_a, sem_a)
    B = (idx_b, xlr_b, xrr_b, sem_b)

    def issue(ci, buf):
      idxb, xlb, xrb, sem = buf
      gci = wid * n_chunks + ci
      pltpu.sync_copy(idx_hbm.at[gci], idxb)
      pltpu.async_copy(xl_hbm.at[idxb.at[0]], xlb, sem)
      pltpu.async_copy(xr_hbm.at[idxb.at[1]], xrb, sem)

    def wait_gather(buf):
      idxb, xlb, xrb, sem = buf
      pltpu.make_async_copy(xl_hbm.at[idxb.at[0]], xlb, sem).wait()
      pltpu.make_async_copy(xr_hbm.at[idxb.at[1]], xrb, sem).wait()

    def process(buf):
      idxb, xlb, xrb, _ = buf
      dstb = idxb.at[1]

      def group_body(g, carry2):
        p16 = jnp.zeros((L,), jnp.float32)
        for u in range(L):
          j = g * L + u
          acc = None
          xs = []
          for q in range(D // L):
            a = xlb[j, pl.ds(q * L, L)]
            xs.append(a)
            t = a + xrb[j, pl.ds(q * L, L)]
            t = jnp.maximum(t, 0.2 * t)
            pr = att_q[q] * t
            acc = pr if acc is None else acc + pr
          pb = jnp.exp(jnp.full((L,), jnp.sum(acc), jnp.float32))
          p16 = jnp.where(row16 == u, pb, p16)
          for q in range(D // L):
            xlb[j, pl.ds(q * L, L)] = xs[q] * pb
        plsc.addupdate_scatter(s_v, [idxb[1, pl.ds(g * L, L)]], p16)
        return carry2

      lax.fori_loop(0, CHUNK // L, group_body, 0)
      pltpu.sync_copy(xlb, acc_sh.at[dstb], add=True)

    issue(0, A)

    def pair_body(i, carry):
      issue(2 * i + 1, B)
      wait_gather(A)
      process(A)

      @pl.when(i < n_pairs - 1)
      def _():
        issue(2 * i + 2, A)

      wait_gather(B)
      process(B)
      return carry

    lax.fori_loop(0, n_pairs, pair_body, 0)
    pltpu.sync_copy(s_v, s_hbm.at[wid])
    plsc.subcore_barrier()
    pltpu.sync_copy(acc_sh.at[pl.ds(sid * rpt, rpt)],
                    out_hbm.at[cid, pl.ds(sid * rpt, rpt)])

  return sc_edge


def kernel(edge_index, emb, Wl1, Wr1, att1, b1, Wl2, Wr2, att2, b2):
  n = emb.shape[0]
  e2 = edge_index.shape[1] + n
  n_pad = ((n + BN) // BN) * BN
  per_w = -(-e2 // (NW * 2 * CHUNK)) * 2 * CHUNK
  e_pad = per_w * NW

  loop = jnp.arange(n, dtype=jnp.int32)
  pad_e = e_pad - e2
  src = jnp.concatenate(
      [edge_index[0], loop, jnp.zeros((pad_e,), jnp.int32)])
  dst = jnp.concatenate(
      [edge_index[1], loop, jnp.full((pad_e,), n, jnp.int32)])
  idx_pk = jnp.stack([src.reshape(-1, CHUNK), dst.reshape(-1, CHUNK)], axis=1)
  emb_p = jnp.pad(emb, ((0, n_pad - n), (0, 0)))
  zero_nd = jnp.zeros((n_pad, D), jnp.float32)
  b1r = b1.reshape(1, D)
  b2r = b2.reshape(1, D)
  sc_edge = _make_sc_edge(e_pad, n_pad, per_w)

  xl1, xr1 = _mm2(emb_p, Wl1, Wr1)
  s1, acc1 = sc_edge(xl1, xr1, idx_pk, att1, zero_nd)
  h, xl2, xr2 = _comb_mm2(acc1, s1, b1r, emb_p, Wl2, Wr2)
  s2, acc2 = sc_edge(xl2, xr2, idx_pk, att2, zero_nd)
  y = _final(acc2, s2, b2r, h)

  y = y[:n]
  return (y[:N_USERS], y[N_USERS:])
